# baseline (speedup 1.0000x reference)
import functools

import jax
import jax.numpy as jnp
from jax import lax
from jax.experimental import pallas as pl
from jax.experimental.pallas import tpu as pltpu
from jax.experimental.pallas import tpu_sc as plsc

N = 1048576
NUM_EVENT = 100000
NUM_STATION = 64
REG = 0.1
VEL0 = 6.0
VEL1 = 6.0 / 1.73

NC = 2
NS = 16
NW = NC * NS
PER_W = N // NW
CHUNK = 2048
G_ROWS = CHUNK // 128
N_CHUNKS = PER_W // CHUNK

TCR = 512
TC_GRID = N // 128 // TCR


def _sc_body(ei2d, si2d, ev_tab, st_tab,
             out_ev, out_st,
             ei_v, si_v, ev4_v, st4_v, sp_ev, sp_st, gsem):
    sid = lax.axis_index("s")
    wid = lax.axis_index("c") * NS + sid
    wbase = wid * PER_W

    @pl.when(sid == 0)
    def _():
        pltpu.sync_copy(ev_tab, sp_ev)
        pltpu.sync_copy(st_tab, sp_st)

    plsc.subcore_barrier()

    def chunk_body(g, _):
        base = pl.multiple_of(wbase + g * CHUNK, CHUNK)
        row = pl.multiple_of(base // 128, 8)

        pltpu.sync_copy(ei2d.at[pl.ds(row, G_ROWS)], ei_v)
        pltpu.sync_copy(si2d.at[pl.ds(row, G_ROWS)], si_v)

        def fire(j, _):
            sl = pl.ds(j * 128, 128)
            pltpu.make_async_copy(sp_ev.at[ei_v.at[j]], ev4_v.at[sl],
                                  gsem).start()
            pltpu.make_async_copy(sp_st.at[si_v.at[j]], st4_v.at[sl],
                                  gsem).start()
            return 0

        def drain(j, _):
            sl = pl.ds(j * 128, 128)
            pltpu.make_async_copy(sp_ev.at[ei_v.at[j]], ev4_v.at[sl],
                                  gsem).wait()
            pltpu.make_async_copy(sp_st.at[si_v.at[j]], st4_v.at[sl],
                                  gsem).wait()
            return 0

        lax.fori_loop(0, G_ROWS, fire, 0)
        lax.fori_loop(0, G_ROWS, drain, 0)

        pltpu.sync_copy(ev4_v, out_ev.at[pl.ds(base, CHUNK)])
        pltpu.sync_copy(st4_v, out_st.at[pl.ds(base, CHUNK)])
        return 0

    lax.fori_loop(0, N_CHUNKS, chunk_body, 0)


@functools.partial(
    pl.kernel,
    mesh=plsc.VectorSubcoreMesh(core_axis_name="c", subcore_axis_name="s"),
    out_type=[
        jax.ShapeDtypeStruct((N, 8), jnp.float32),
        jax.ShapeDtypeStruct((N, 8), jnp.float32),
    ],
    scratch_types=[
        pltpu.VMEM((G_ROWS, 128), jnp.int32),
        pltpu.VMEM((G_ROWS, 128), jnp.int32),
        pltpu.VMEM((CHUNK, 8), jnp.float32),
        pltpu.VMEM((CHUNK, 8), jnp.float32),
        pltpu.VMEM_SHARED((NUM_EVENT, 8), jnp.float32),
        pltpu.VMEM_SHARED((2 * NUM_STATION, 8), jnp.float32),
        pltpu.SemaphoreType.DMA,
    ],
    compiler_params=pltpu.CompilerParams(use_tc_tiling_on_sc=False),
)
def _gather_sc(ei2d, si2d, ev_tab, st_tab, out_ev, out_st, *scratch):
    _sc_body(ei2d, si2d, ev_tab, st_tab, out_ev, out_st, *scratch)


def _tc_body(evr_ref, str_ref, m1_ref, ptm_ref, pw_ref,
             pred_ref, lp_ref, acc_ref, sel_ref):
    i = pl.program_id(0)

    @pl.when(i == 0)
    def _():
        jj = lax.broadcasted_iota(jnp.int32, (1024, 256), 0)
        kk = lax.broadcasted_iota(jnp.int32, (1024, 256), 1)
        ss = ((jj // 8 == kk % 128) & (jj % 8 < 3) & (kk < 128))
        s3 = ((jj // 8 == kk % 128) & (jj % 8 == 3) & (kk >= 128))
        sel_ref[...] = (ss | s3).astype(jnp.float32)

    selsum = sel_ref[...][:, 0:128]
    sel3 = sel_ref[...][:, 128:256]
    evr = evr_ref[...]
    str_ = str_ref[...]

    diff = evr - str_
    d2 = jnp.dot(diff * diff, selsum,
                 preferred_element_type=jnp.float32)
    etdt = jnp.dot(evr + str_, sel3,
                   preferred_element_type=jnp.float32)
    dtv = jnp.dot(str_, sel3, preferred_element_type=jnp.float32)

    m1 = m1_ref[...]
    ptm = ptm_ref[...]
    w = pw_ref[...]

    dist = jnp.sqrt(d2)
    m0 = 1.0 - m1
    vel = VEL0 + (VEL1 - VEL0) * m1
    tt = dist / vel
    pred = etdt + tt
    pred_ref[...] = pred

    resid = pred - ptm
    ar = jnp.abs(resid)
    hub = jnp.where(ar < 1.0, 0.5 * resid * resid, ar - 0.5)
    contrib = hub * w + REG * jnp.abs(dtv)

    @pl.when(i == 0)
    def _():
        acc_ref[...] = jnp.zeros((4 * TCR, 128), jnp.float32)

    acc_ref[pl.ds(0, TCR), :] += contrib * m0
    acc_ref[pl.ds(TCR, TCR), :] += contrib * m1
    acc_ref[pl.ds(2 * TCR, TCR), :] += m0
    acc_ref[pl.ds(3 * TCR, TCR), :] += m1

    @pl.when(i == TC_GRID - 1)
    def _():
        lp_ref[...] = acc_ref[...]


_tc_call = pl.pallas_call(
    _tc_body,
    grid=(TC_GRID,),
    in_specs=[
        pl.BlockSpec((TCR, 1024), lambda i: (i, 0)),
        pl.BlockSpec((TCR, 1024), lambda i: (i, 0)),
        pl.BlockSpec((TCR, 128), lambda i: (i, 0)),
        pl.BlockSpec((TCR, 128), lambda i: (i, 0)),
        pl.BlockSpec((TCR, 128), lambda i: (i, 0)),
    ],
    out_specs=[
        pl.BlockSpec((TCR, 128), lambda i: (i, 0)),
        pl.BlockSpec((4 * TCR, 128), lambda i: (0, 0)),
    ],
    out_shape=[
        jax.ShapeDtypeStruct((N // 128, 128), jnp.float32),
        jax.ShapeDtypeStruct((4 * TCR, 128), jnp.float32),
    ],
    scratch_shapes=[pltpu.VMEM((4 * TCR, 128), jnp.float32),
                    pltpu.VMEM((1024, 256), jnp.float32)],
)


def kernel(station_index, event_index, phase_type, phase_time, phase_weight,
           event_loc_w, event_time_w, station_loc_w, station_dt_w):
    st_i = station_index.astype(jnp.int32)
    ph_t = phase_type.astype(jnp.int32)
    ei2d = event_index.astype(jnp.int32).reshape(N // 128, 128)
    si2d = (st_i + st_i + ph_t).reshape(N // 128, 128)

    ev_tab = jnp.concatenate(
        [event_loc_w, event_time_w,
         jnp.zeros((NUM_EVENT, 4), jnp.float32)], axis=1)
    st_tab = jnp.concatenate(
        [jnp.repeat(station_loc_w, 2, axis=0),
         station_dt_w.reshape(2 * NUM_STATION, 1),
         jnp.zeros((2 * NUM_STATION, 4), jnp.float32)], axis=1)

    rows_ev, rows_st = _gather_sc(ei2d, si2d, ev_tab, st_tab)

    m1_2 = ph_t.astype(jnp.float32).reshape(N // 128, 128)
    ptm2 = phase_time.reshape(N // 128, 128)
    pw2 = phase_weight.reshape(N // 128, 128)
    evr2 = rows_ev.reshape(N // 128, 1024)
    str2 = rows_st.reshape(N // 128, 1024)

    pred2, lp = _tc_call(evr2, str2, m1_2, ptm2, pw2)

    l0 = jnp.sum(lp[0:TCR])
    l1 = jnp.sum(lp[TCR:2 * TCR])
    c0 = jnp.maximum(jnp.sum(lp[2 * TCR:3 * TCR]), 1.0)
    c1 = jnp.maximum(jnp.sum(lp[3 * TCR:4 * TCR]), 1.0)
    loss = l0 / c0 + l1 / c1
    return pred2.reshape(N, 1), loss


# speedup vs baseline: 1.0104x; 1.0104x over previous
import functools

import jax
import jax.numpy as jnp
from jax import lax
from jax.experimental import pallas as pl
from jax.experimental.pallas import tpu as pltpu
from jax.experimental.pallas import tpu_sc as plsc

N = 1048576
NUM_EVENT = 100000
NUM_STATION = 64
REG = 0.1
VEL0 = 6.0
VEL1 = 6.0 / 1.73

NC = 2
NS = 16
NW = NC * NS
PER_W = N // NW
CHUNK = 2048
G_ROWS = CHUNK // 128
N_CHUNKS = PER_W // CHUNK

TCR = 1024
TC_GRID = N // 128 // TCR


def _sc_body(ei2d, si2d, ev_tab, st_tab,
             out_ev, out_st,
             ei_v, si_v, ev4_v, st4_v, sp_ev, sp_st, gsem):
    sid = lax.axis_index("s")
    wid = lax.axis_index("c") * NS + sid
    wbase = wid * PER_W

    @pl.when(sid == 0)
    def _():
        pltpu.sync_copy(ev_tab, sp_ev)
        pltpu.sync_copy(st_tab, sp_st)

    plsc.subcore_barrier()

    def chunk_body(g, _):
        base = pl.multiple_of(wbase + g * CHUNK, CHUNK)
        row = pl.multiple_of(base // 128, 8)

        pltpu.sync_copy(ei2d.at[pl.ds(row, G_ROWS)], ei_v)
        pltpu.sync_copy(si2d.at[pl.ds(row, G_ROWS)], si_v)

        def fire(j, _):
            sl = pl.ds(j * 128, 128)
            pltpu.make_async_copy(sp_ev.at[ei_v.at[j]], ev4_v.at[sl],
                                  gsem).start()
            pltpu.make_async_copy(sp_st.at[si_v.at[j]], st4_v.at[sl],
                                  gsem).start()
            return 0

        def drain(j, _):
            sl = pl.ds(j * 128, 128)
            pltpu.make_async_copy(sp_ev.at[ei_v.at[j]], ev4_v.at[sl],
                                  gsem).wait()
            pltpu.make_async_copy(sp_st.at[si_v.at[j]], st4_v.at[sl],
                                  gsem).wait()
            return 0

        lax.fori_loop(0, G_ROWS, fire, 0)
        lax.fori_loop(0, G_ROWS, drain, 0)

        pltpu.sync_copy(ev4_v, out_ev.at[pl.ds(base, CHUNK)])
        pltpu.sync_copy(st4_v, out_st.at[pl.ds(base, CHUNK)])
        return 0

    lax.fori_loop(0, N_CHUNKS, chunk_body, 0)


@functools.partial(
    pl.kernel,
    mesh=plsc.VectorSubcoreMesh(core_axis_name="c", subcore_axis_name="s"),
    out_type=[
        jax.ShapeDtypeStruct((N, 8), jnp.float32),
        jax.ShapeDtypeStruct((N, 8), jnp.float32),
    ],
    scratch_types=[
        pltpu.VMEM((G_ROWS, 128), jnp.int32),
        pltpu.VMEM((G_ROWS, 128), jnp.int32),
        pltpu.VMEM((CHUNK, 8), jnp.float32),
        pltpu.VMEM((CHUNK, 8), jnp.float32),
        pltpu.VMEM_SHARED((NUM_EVENT, 8), jnp.float32),
        pltpu.VMEM_SHARED((2 * NUM_STATION, 8), jnp.float32),
        pltpu.SemaphoreType.DMA,
    ],
    compiler_params=pltpu.CompilerParams(use_tc_tiling_on_sc=False),
)
def _gather_sc(ei2d, si2d, ev_tab, st_tab, out_ev, out_st, *scratch):
    _sc_body(ei2d, si2d, ev_tab, st_tab, out_ev, out_st, *scratch)


def _tc_body(evr_ref, str_ref, m1_ref, ptm_ref, pw_ref,
             pred_ref, lp_ref, acc_ref, sel_ref):
    i = pl.program_id(0)

    @pl.when(i == 0)
    def _():
        jj = lax.broadcasted_iota(jnp.int32, (1024, 256), 0)
        kk = lax.broadcasted_iota(jnp.int32, (1024, 256), 1)
        ss = ((jj // 8 == kk % 128) & (jj % 8 < 3) & (kk < 128))
        s3 = ((jj // 8 == kk % 128) & (jj % 8 == 3) & (kk >= 128))
        sel_ref[...] = (ss | s3).astype(jnp.float32)

    selsum = sel_ref[...][:, 0:128]
    sel3 = sel_ref[...][:, 128:256]
    evr = evr_ref[...]
    str_ = str_ref[...]

    diff = evr - str_
    d2 = jnp.dot(diff * diff, selsum,
                 preferred_element_type=jnp.float32)
    etdt = jnp.dot(evr + str_, sel3,
                   preferred_element_type=jnp.float32)
    dtv = jnp.dot(str_, sel3, preferred_element_type=jnp.float32)

    m1 = m1_ref[...]
    ptm = ptm_ref[...]
    w = pw_ref[...]

    dist = jnp.sqrt(d2)
    m0 = 1.0 - m1
    vel = VEL0 + (VEL1 - VEL0) * m1
    tt = dist / vel
    pred = etdt + tt
    pred_ref[...] = pred

    resid = pred - ptm
    ar = jnp.abs(resid)
    hub = jnp.where(ar < 1.0, 0.5 * resid * resid, ar - 0.5)
    contrib = hub * w + REG * jnp.abs(dtv)

    @pl.when(i == 0)
    def _():
        acc_ref[...] = jnp.zeros((4 * TCR, 128), jnp.float32)

    acc_ref[pl.ds(0, TCR), :] += contrib * m0
    acc_ref[pl.ds(TCR, TCR), :] += contrib * m1
    acc_ref[pl.ds(2 * TCR, TCR), :] += m0
    acc_ref[pl.ds(3 * TCR, TCR), :] += m1

    @pl.when(i == TC_GRID - 1)
    def _():
        lp_ref[...] = acc_ref[...]


_tc_call = pl.pallas_call(
    _tc_body,
    grid=(TC_GRID,),
    in_specs=[
        pl.BlockSpec((TCR, 1024), lambda i: (i, 0)),
        pl.BlockSpec((TCR, 1024), lambda i: (i, 0)),
        pl.BlockSpec((TCR, 128), lambda i: (i, 0)),
        pl.BlockSpec((TCR, 128), lambda i: (i, 0)),
        pl.BlockSpec((TCR, 128), lambda i: (i, 0)),
    ],
    out_specs=[
        pl.BlockSpec((TCR, 128), lambda i: (i, 0)),
        pl.BlockSpec((4 * TCR, 128), lambda i: (0, 0)),
    ],
    out_shape=[
        jax.ShapeDtypeStruct((N // 128, 128), jnp.float32),
        jax.ShapeDtypeStruct((4 * TCR, 128), jnp.float32),
    ],
    scratch_shapes=[pltpu.VMEM((4 * TCR, 128), jnp.float32),
                    pltpu.VMEM((1024, 256), jnp.float32)],
)


def kernel(station_index, event_index, phase_type, phase_time, phase_weight,
           event_loc_w, event_time_w, station_loc_w, station_dt_w):
    st_i = station_index.astype(jnp.int32)
    ph_t = phase_type.astype(jnp.int32)
    ei2d = event_index.astype(jnp.int32).reshape(N // 128, 128)
    si2d = (st_i + st_i + ph_t).reshape(N // 128, 128)

    ev_tab = jnp.concatenate(
        [event_loc_w, event_time_w,
         jnp.zeros((NUM_EVENT, 4), jnp.float32)], axis=1)
    st_tab = jnp.concatenate(
        [jnp.repeat(station_loc_w, 2, axis=0),
         station_dt_w.reshape(2 * NUM_STATION, 1),
         jnp.zeros((2 * NUM_STATION, 4), jnp.float32)], axis=1)

    rows_ev, rows_st = _gather_sc(ei2d, si2d, ev_tab, st_tab)

    m1_2 = ph_t.astype(jnp.float32).reshape(N // 128, 128)
    ptm2 = phase_time.reshape(N // 128, 128)
    pw2 = phase_weight.reshape(N // 128, 128)
    evr2 = rows_ev.reshape(N // 128, 1024)
    str2 = rows_st.reshape(N // 128, 1024)

    pred2, lp = _tc_call(evr2, str2, m1_2, ptm2, pw2)

    l0 = jnp.sum(lp[0:TCR])
    l1 = jnp.sum(lp[TCR:2 * TCR])
    c0 = jnp.maximum(jnp.sum(lp[2 * TCR:3 * TCR]), 1.0)
    c1 = jnp.maximum(jnp.sum(lp[3 * TCR:4 * TCR]), 1.0)
    loss = l0 / c0 + l1 / c1
    return pred2.reshape(N, 1), loss
